# baseline (speedup 1.0000x reference)
import functools

import jax
import jax.numpy as jnp
from jax import lax
from jax.experimental import pallas as pl
from jax.experimental.pallas import tpu as pltpu
from jax.experimental.pallas import tpu_sc as plsc

_N = 256
_IN_F = 256
_HID = 128
_HEADS = 4
_OUT_LEN = 32
_FC = _N * _OUT_LEN
_FC_BLK = 256
_FC_SPLIT = 4

_SC_ROWS = 2048
_SC_START = _FC - _SC_ROWS
_SC_WORKERS = 32
_SC_RPW = _SC_ROWS // _SC_WORKERS
_SC_BR = 4
_SC_NB = _SC_RPW // _SC_BR
_LANES = 16
_CHUNKS = _FC // _LANES


def _row_softmax_attention(h, att_src, att_dst):
    dn = (((1,), (1,)), ((), ()))
    a_src_row = jax.lax.dot_general(att_src, h, dn,
                                    preferred_element_type=jnp.float32)
    a_dst_col = jax.lax.dot_general(h, att_dst, dn,
                                    preferred_element_type=jnp.float32)
    e = a_dst_col + a_src_row
    e = jnp.where(e >= 0, e, 0.2 * e)
    m = jnp.max(e, axis=1, keepdims=True)
    p = jnp.exp(e - m)
    s = jnp.sum(p, axis=1, keepdims=True)
    alpha = p / (s + 1e-16)
    return jnp.dot(alpha, h, preferred_element_type=jnp.float32)


def _gat_kernel(x_ref, w1_ref, as1_ref, ad1_ref, b1_ref,
                w2_ref, as2_ref, ad2_ref, b2_ref, out_ref):
    h1 = jnp.dot(x_ref[...], w1_ref[...],
                 preferred_element_type=jnp.float32)
    outs = []
    for hd in range(_HEADS):
        hh = h1[:, hd * _HID:(hd + 1) * _HID]
        outs.append(_row_softmax_attention(hh,
                                           as1_ref[hd:hd + 1, :],
                                           ad1_ref[hd:hd + 1, :]))
    h = jnp.concatenate(outs, axis=1) + b1_ref[...]
    h = jnp.maximum(h, 0.0)

    h2 = jnp.dot(h, w2_ref[...], preferred_element_type=jnp.float32)
    out2 = _row_softmax_attention(h2, as2_ref[...], ad2_ref[...])
    out_ref[...] = jnp.maximum(out2 + b2_ref[...], 0.0)


def _fc_kernel(h_ref, bfc_ref, *wfc_refs_and_out):
    wfc_refs = wfc_refs_and_out[:-1]
    out_ref = wfc_refs_and_out[-1]
    dn = (((1,), (1,)), ((), ()))
    ys = [jax.lax.dot_general(h_ref[...], w[...], dn,
                              preferred_element_type=jnp.float32)
          for w in wfc_refs]
    out_ref[...] = jnp.concatenate(ys, axis=1) + bfc_ref[...]


def _sc_fc_kernel(wfc_hbm, h_hbm, out_hbm, h_buf, w_buf0, w_buf1, y_buf,
                  sem0, sem1):
    wid = lax.axis_index("s") * 2 + lax.axis_index("c")
    row0 = _SC_START + wid * _SC_RPW

    pltpu.sync_copy(h_hbm, h_buf)

    bufs = (w_buf0, w_buf1)
    sems = (sem0, sem1)

    def start(b):
        return pltpu.async_copy(
            wfc_hbm.at[pl.ds(row0 + b * _SC_BR, _SC_BR)],
            bufs[b % 2], sems[b % 2])

    pending = start(0)
    for b in range(_SC_NB):
        pending.wait()
        if b + 1 < _SC_NB:
            pending = start(b + 1)
        wb = bufs[b % 2]

        def chunk(c, accs):
            off = pl.multiple_of(c * _LANES, _LANES)
            hv = h_buf[pl.ds(off, _LANES)]
            return tuple(accs[r] + wb[r, pl.ds(off, _LANES)] * hv
                         for r in range(_SC_BR))

        zeros = tuple(jnp.zeros((_LANES,), jnp.float32)
                      for _ in range(_SC_BR))
        accs = lax.fori_loop(0, _CHUNKS, chunk, zeros)
        for r in range(_SC_BR):
            y_buf[b * _SC_BR + r] = accs[r]

    pltpu.sync_copy(y_buf, out_hbm.at[pl.ds(wid * _SC_RPW, _SC_RPW)])


@functools.partial(
    pl.kernel,
    mesh=plsc.VectorSubcoreMesh(core_axis_name="c", subcore_axis_name="s"),
    out_type=jax.ShapeDtypeStruct((_SC_ROWS, _LANES), jnp.float32),
    scratch_types=[
        pltpu.VMEM((_FC,), jnp.float32),
        pltpu.VMEM((_SC_BR, _FC), jnp.float32),
        pltpu.VMEM((_SC_BR, _FC), jnp.float32),
        pltpu.VMEM((_SC_RPW, _LANES), jnp.float32),
        pltpu.SemaphoreType.DMA,
        pltpu.SemaphoreType.DMA,
    ],
)
def _sc_fc(wfc_hbm, h_hbm, out_hbm, h_buf, w_buf0, w_buf1, y_buf, sem0, sem1):
    _sc_fc_kernel(wfc_hbm, h_hbm, out_hbm, h_buf, w_buf0, w_buf1, y_buf,
                  sem0, sem1)


def kernel(x, edge_index, W1, att_src1, att_dst1, b1,
           W2, att_src2, att_dst2, b2, Wfc, bfc):
    del edge_index

    h2 = pl.pallas_call(
        _gat_kernel,
        out_shape=jax.ShapeDtypeStruct((_N, _OUT_LEN), jnp.float32),
    )(x, W1, att_src1, att_dst1, b1.reshape(1, _HEADS * _HID),
      W2, att_src2, att_dst2, b2.reshape(1, _OUT_LEN))

    h2row = h2.reshape(1, _FC)
    sub = _FC_BLK // _FC_SPLIT
    wfc_specs = [
        pl.BlockSpec((sub, _FC), lambda i, j=j: (_FC_SPLIT * i + j, 0))
        for j in range(_FC_SPLIT)
    ]
    y_tc = pl.pallas_call(
        _fc_kernel,
        grid=(_SC_START // _FC_BLK,),
        in_specs=[
            pl.BlockSpec((1, _FC), lambda i: (0, 0)),
            pl.BlockSpec((1, _FC_BLK), lambda i: (0, i)),
        ] + wfc_specs,
        out_specs=pl.BlockSpec((1, _FC_BLK), lambda i: (0, i)),
        out_shape=jax.ShapeDtypeStruct((1, _SC_START), jnp.float32),
    )(h2row, bfc.reshape(1, _FC), *([Wfc] * _FC_SPLIT))

    part = _sc_fc(Wfc, h2.reshape(_FC))
    y_sc = part.sum(axis=1) + bfc.reshape(_FC)[_SC_START:]

    y = jnp.concatenate([y_tc.reshape(_SC_START), y_sc])
    return y.reshape(1, _N, _OUT_LEN)


# speedup vs baseline: 1.2275x; 1.2275x over previous
import jax
import jax.numpy as jnp
from jax.experimental import pallas as pl

_N = 256
_IN_F = 256
_HID = 128
_HEADS = 4
_OUT_LEN = 32
_FC = _N * _OUT_LEN
_FC_BLK = 256
_FC_SPLIT = 4


def _row_softmax_attention(h, att_src, att_dst):
    dn = (((1,), (1,)), ((), ()))
    a_src_row = jax.lax.dot_general(att_src, h, dn,
                                    preferred_element_type=jnp.float32)
    a_dst_col = jax.lax.dot_general(h, att_dst, dn,
                                    preferred_element_type=jnp.float32)
    e = a_dst_col + a_src_row
    e = jnp.where(e >= 0, e, 0.2 * e)
    m = jnp.max(e, axis=1, keepdims=True)
    p = jnp.exp(e - m)
    s = jnp.sum(p, axis=1, keepdims=True)
    alpha = p / (s + 1e-16)
    return jnp.dot(alpha, h, preferred_element_type=jnp.float32)


def _gat_kernel(x_ref, w1_ref, as1_ref, ad1_ref, b1_ref,
                w2_ref, as2_ref, ad2_ref, b2_ref, out_ref):
    h1 = jnp.dot(x_ref[...], w1_ref[...],
                 preferred_element_type=jnp.float32)
    outs = []
    for hd in range(_HEADS):
        hh = h1[:, hd * _HID:(hd + 1) * _HID]
        outs.append(_row_softmax_attention(hh,
                                           as1_ref[hd:hd + 1, :],
                                           ad1_ref[hd:hd + 1, :]))
    h = jnp.concatenate(outs, axis=1) + b1_ref[...]
    h = jnp.maximum(h, 0.0)

    h2 = jnp.dot(h, w2_ref[...], preferred_element_type=jnp.float32)
    out2 = _row_softmax_attention(h2, as2_ref[...], ad2_ref[...])
    out_ref[...] = jnp.maximum(out2 + b2_ref[...], 0.0)


def _fc_kernel(h_ref, bfc_ref, *wfc_refs_and_out):
    wfc_refs = wfc_refs_and_out[:-1]
    out_ref = wfc_refs_and_out[-1]
    dn = (((1,), (1,)), ((), ()))
    ys = [jax.lax.dot_general(h_ref[...], w[...], dn,
                              preferred_element_type=jnp.float32)
          for w in wfc_refs]
    out_ref[...] = jnp.concatenate(ys, axis=1) + bfc_ref[...]


def kernel(x, edge_index, W1, att_src1, att_dst1, b1,
           W2, att_src2, att_dst2, b2, Wfc, bfc):
    del edge_index

    h2 = pl.pallas_call(
        _gat_kernel,
        out_shape=jax.ShapeDtypeStruct((_N, _OUT_LEN), jnp.float32),
    )(x, W1, att_src1, att_dst1, b1.reshape(1, _HEADS * _HID),
      W2, att_src2, att_dst2, b2.reshape(1, _OUT_LEN))

    h2row = h2.reshape(1, _FC)
    sub = _FC_BLK // _FC_SPLIT
    wfc_specs = [
        pl.BlockSpec((sub, _FC), lambda i, j=j: (_FC_SPLIT * i + j, 0))
        for j in range(_FC_SPLIT)
    ]
    y = pl.pallas_call(
        _fc_kernel,
        grid=(_FC // _FC_BLK,),
        in_specs=[
            pl.BlockSpec((1, _FC), lambda i: (0, 0)),
            pl.BlockSpec((1, _FC_BLK), lambda i: (0, i)),
        ] + wfc_specs,
        out_specs=pl.BlockSpec((1, _FC_BLK), lambda i: (0, i)),
        out_shape=jax.ShapeDtypeStruct((1, _FC), jnp.float32),
    )(h2row, bfc.reshape(1, _FC), *([Wfc] * _FC_SPLIT))

    return y.reshape(1, _N, _OUT_LEN)
